# baseline (speedup 1.0000x reference)
import functools

import numpy as np
import jax
import jax.numpy as jnp
from jax import lax
from jax.experimental import pallas as pl
from jax.experimental.pallas import tpu as pltpu
from jax.experimental.pallas import tpu_sc as plsc

_NBINS = 100
_CUTOFF = 0.35
_NA = 500
_W = _CUTOFF / (_NBINS - 1)
_INVW = (_NBINS - 1) / _CUTOFF
_J = 6
_NH = 128
_R2T = (_CUTOFF + _J * _W) ** 2
_NW = 32
_BUF = 8448

_mesh = plsc.VectorSubcoreMesh(core_axis_name="c", subcore_axis_name="s")


@functools.partial(
    pl.kernel,
    out_type=jax.ShapeDtypeStruct((_NW * _NH,), jnp.float32),
    mesh=_mesh,
    compiler_params=pltpu.CompilerParams(needs_layout_passes=False),
    scratch_types=[
        pltpu.VMEM((3 * _NA * 2,), jnp.float32),
        pltpu.VMEM((_BUF,), jnp.float32),
        pltpu.VMEM((16 * _NH,), jnp.float32),
        pltpu.VMEM((_NH,), jnp.float32),
    ],
)
def _sc_hist(coords_hbm, out_hbm, cvm, buf, hist, outv):
    wid = lax.axis_index("s") * 2 + lax.axis_index("c")
    pltpu.sync_copy(coords_hbm, cvm)
    iota = lax.iota(jnp.int32, 16)
    iota3 = iota * 3
    zero16 = jnp.zeros((16,), jnp.float32)
    for c in range(16 * _NH // 16):
        hist[pl.ds(c * 16, 16)] = zero16

    def wrap_sq(d):
        w = d - (2.0 * d).astype(jnp.int32).astype(jnp.float32)
        return w * w

    def one_batch(b, cursor):
        base = b * 3 * _NA
        nrows = (_NA - 1 - wid) // _NW + 1

        def row_body(ri, cur):
            i = wid + _NW * ri
            civ = jnp.full((16,), base + 3 * i, jnp.int32)
            xi = plsc.load_gather(cvm, [civ])
            yi = plsc.load_gather(cvm, [civ + 1])
            zi = plsc.load_gather(cvm, [civ + 2])
            nj = (i + 15) // 16

            def jv_body(jv, cur2):
                jidx = jv * 16 + iota
                ji = base + jv * 48 + iota3
                dsq = wrap_sq(xi - plsc.load_gather(cvm, [ji]))
                dsq = dsq + wrap_sq(yi - plsc.load_gather(cvm, [ji + 1]))
                dsq = dsq + wrap_sq(zi - plsc.load_gather(cvm, [ji + 2]))
                m = (dsq < _R2T) & (dsq != 0.0) & (jidx < i)
                pos = plsc.cumsum(m.astype(jnp.int32))
                plsc.store_scatter(buf, [cur2 + (pos - 1)], dsq, mask=m)
                return cur2 + plsc.all_reduce_population_count(m)

            return lax.fori_loop(0, nj, jv_body, cur)

        return lax.fori_loop(0, nrows, row_body, cursor)

    nvec = one_batch(0, jnp.zeros((16,), jnp.int32))
    nvec = one_batch(1, nvec)
    n = nvec[0]

    nv = (n + 15) // 16
    ratio_c = [float(np.exp(-(j + 0.5))) for j in range(-_J, _J)]

    def pv(kv, carry):
        off = kv * 16
        dsq = buf[pl.ds(off, 16)]
        valid = (off + iota) < n
        bits = plsc.bitcast(dsq, jnp.int32)
        y = plsc.bitcast(
            jnp.int32(0x5F3759DF) - lax.shift_right_logical(bits, 1),
            jnp.float32)
        for _ in range(3):
            y = y * (1.5 - 0.5 * dsq * y * y)
        t = dsq * y * _INVW
        i0 = (t + 0.5).astype(jnp.int32)
        i0 = jnp.minimum(jnp.maximum(i0, 0), _NBINS + _J)
        f = t - i0.astype(jnp.float32)
        ef = jnp.exp(f)
        wv = jnp.exp(-0.5 * (f + _J) * (f + _J))
        base_idx = iota * _NH + i0
        for jj in range(2 * _J + 1):
            plsc.addupdate_scatter(hist, [base_idx + jj], wv, mask=valid)
            if jj < 2 * _J:
                wv = wv * ef * ratio_c[jj]
        return carry

    lax.fori_loop(0, nv, pv, jnp.int32(0))

    for c in range(8):
        acc = hist[pl.ds(c * 16, 16)]
        for r in range(1, 16):
            acc = acc + hist[pl.ds(r * _NH + c * 16, 16)]
        outv[pl.ds(c * 16, 16)] = acc
    pltpu.sync_copy(outv, out_hbm.at[pl.ds(wid * _NH, _NH)])


def kernel(xyz):
    coords = xyz.reshape(-1)
    part = _sc_hist(coords).reshape(_NW, _NH)
    count = part.sum(axis=0)[_J:_J + _NBINS]
    bins = jnp.linspace(0.0, _CUTOFF, _NBINS + 1)
    vol_bins = 4.0 * np.pi / 3.0 * (bins[1:] ** 3 - bins[:-1] ** 3)
    norm = count.sum()
    count = count / norm
    V = 4.0 / 3.0 * np.pi * _CUTOFF ** 3
    rdf_out = count / (vol_bins / V)
    return (count, bins, rdf_out)


# speedup vs baseline: 1.0342x; 1.0342x over previous
import functools

import numpy as np
import jax
import jax.numpy as jnp
from jax import lax
from jax.experimental import pallas as pl
from jax.experimental.pallas import tpu as pltpu
from jax.experimental.pallas import tpu_sc as plsc

_NBINS = 100
_CUTOFF = 0.35
_NA = 500
_W = _CUTOFF / (_NBINS - 1)
_INVW = (_NBINS - 1) / _CUTOFF
_J = 6
_NH = 128
_R2T = (_CUTOFF + _J * _W) ** 2
_NW = 32
_BUF = 8448

_mesh = plsc.VectorSubcoreMesh(core_axis_name="c", subcore_axis_name="s")


@functools.partial(
    pl.kernel,
    out_type=jax.ShapeDtypeStruct((_NW * _NH,), jnp.float32),
    mesh=_mesh,
    compiler_params=pltpu.CompilerParams(needs_layout_passes=False),
    scratch_types=[
        pltpu.VMEM((3 * _NA * 2,), jnp.float32),
        pltpu.VMEM((_BUF,), jnp.float32),
        pltpu.VMEM((16 * _NH,), jnp.float32),
        pltpu.VMEM((_NH,), jnp.float32),
    ],
)
def _sc_hist(coords_hbm, out_hbm, cvm, buf, hist, outv):
    wid = lax.axis_index("s") * 2 + lax.axis_index("c")
    pltpu.sync_copy(coords_hbm, cvm)
    iota = lax.iota(jnp.int32, 16)
    iota3 = iota * 3
    zero16 = jnp.zeros((16,), jnp.float32)
    for c in range(16 * _NH // 16):
        hist[pl.ds(c * 16, 16)] = zero16

    def wrap_sq(d):
        w = d - (2.0 * d).astype(jnp.int32).astype(jnp.float32)
        return w * w

    def one_batch(b, cursor):
        base = b * 3 * _NA
        nrows = (_NA - 1 - wid) // _NW + 1

        def row_body(ri, cur):
            i = wid + _NW * ri
            civ = jnp.full((16,), base + 3 * i, jnp.int32)
            xi = plsc.load_gather(cvm, [civ])
            yi = plsc.load_gather(cvm, [civ + 1])
            zi = plsc.load_gather(cvm, [civ + 2])
            nj = (i + 15) // 16

            def jv_body(jv, cur2):
                jidx = jv * 16 + iota
                ji = base + jv * 48 + iota3
                dsq = wrap_sq(xi - plsc.load_gather(cvm, [ji]))
                dsq = dsq + wrap_sq(yi - plsc.load_gather(cvm, [ji + 1]))
                dsq = dsq + wrap_sq(zi - plsc.load_gather(cvm, [ji + 2]))
                m = (dsq < _R2T) & (dsq != 0.0) & (jidx < i)
                pos = plsc.cumsum(m.astype(jnp.int32))
                plsc.store_scatter(buf, [cur2 + (pos - 1)], dsq, mask=m)
                return cur2 + plsc.all_reduce_population_count(m)

            return lax.fori_loop(0, nj, jv_body, cur)

        return lax.fori_loop(0, nrows, row_body, cursor)

    nvec = one_batch(0, jnp.zeros((16,), jnp.int32))
    nvec = one_batch(1, nvec)
    n = nvec[0]

    nv = (n + 15) // 16
    ratio_c = [float(np.exp(-(j + 0.5))) for j in range(-_J, _J)]

    def pv(kv, carry):
        off = kv * 16
        dsq = buf[pl.ds(off, 16)]
        valid = (off + iota) < n
        bits = plsc.bitcast(dsq, jnp.int32)
        y = plsc.bitcast(
            jnp.int32(0x5F3759DF) - lax.shift_right_logical(bits, 1),
            jnp.float32)
        for _ in range(3):
            y = y * (1.5 - 0.5 * dsq * y * y)
        t = dsq * y * _INVW
        i0 = (t + 0.5).astype(jnp.int32)
        i0 = jnp.minimum(jnp.maximum(i0, 0), _NBINS + _J)
        f = t - i0.astype(jnp.float32)
        base_idx = iota * _NH + i0
        for jj in range(2 * _J + 1):
            a = f + float(_J - jj)
            wv = jnp.exp(-0.5 * a * a)
            plsc.addupdate_scatter(hist, [base_idx + jj], wv, mask=valid)
        return carry

    lax.fori_loop(0, nv, pv, jnp.int32(0))

    for c in range(8):
        acc = hist[pl.ds(c * 16, 16)]
        for r in range(1, 16):
            acc = acc + hist[pl.ds(r * _NH + c * 16, 16)]
        outv[pl.ds(c * 16, 16)] = acc
    pltpu.sync_copy(outv, out_hbm.at[pl.ds(wid * _NH, _NH)])


def kernel(xyz):
    coords = xyz.reshape(-1)
    part = _sc_hist(coords).reshape(_NW, _NH)
    count = part.sum(axis=0)[_J:_J + _NBINS]
    bins = jnp.linspace(0.0, _CUTOFF, _NBINS + 1)
    vol_bins = 4.0 * np.pi / 3.0 * (bins[1:] ** 3 - bins[:-1] ** 3)
    norm = count.sum()
    count = count / norm
    V = 4.0 / 3.0 * np.pi * _CUTOFF ** 3
    rdf_out = count / (vol_bins / V)
    return (count, bins, rdf_out)
